# baseline (device time: 15889 ns/iter reference)
import jax
import jax.numpy as jnp
from jax import lax
from jax.experimental import pallas as pl
from jax.experimental.pallas import tpu as pltpu

N_DEV = 4


def kernel(x):
    m_per, n = x.shape
    half = m_per // 2

    def body(x_ref, out_ref, send_sems, recv_sems):
        me = lax.axis_index("i")
        right = lax.rem(me + 1, N_DEV)
        left = lax.rem(me + N_DEV - 1, N_DEV)
        diag = lax.rem(me + 2, N_DEV)

        barrier = pltpu.get_barrier_semaphore()
        for nbr in (left, right):
            pl.semaphore_signal(
                barrier, inc=1,
                device_id=(nbr,), device_id_type=pl.DeviceIdType.MESH,
            )
        pl.semaphore_wait(barrier, 2)

        def sl(origin, half_idx):
            return out_ref.at[pl.ds(origin * m_per + half_idx * half, half), :]

        def make(idx, origin, half_idx, target):
            return pltpu.make_async_remote_copy(
                src_ref=sl(origin, half_idx),
                dst_ref=sl(origin, half_idx),
                send_sem=send_sems.at[idx],
                recv_sem=recv_sems.at[idx],
                device_id=(target,),
                device_id_type=pl.DeviceIdType.MESH,
            )

        s_top_r = make(0, me, 0, right)
        s_bot_r = make(1, me, 1, right)
        s_bot_l = make(2, me, 1, left)
        s_top_l = make(3, me, 0, left)
        s_fwd_r = make(4, left, 0, right)
        s_fwd_l = make(5, right, 1, left)

        r_top_left = make(0, left, 0, left)
        r_bot_left = make(1, left, 1, left)
        r_bot_right = make(2, right, 1, right)
        r_top_right = make(3, right, 0, right)
        r_top_diag = make(4, diag, 0, left)
        r_bot_diag = make(5, diag, 1, right)

        sl(me, 0)[...] = x_ref[pl.ds(0, half), :].astype(out_ref.dtype)
        s_top_r.start()
        sl(me, 1)[...] = x_ref[pl.ds(half, half), :].astype(out_ref.dtype)
        s_bot_l.start()
        s_bot_r.start()
        s_top_l.start()

        r_top_left.wait_recv()
        s_fwd_r.start()
        r_bot_right.wait_recv()
        s_fwd_l.start()

        r_bot_left.wait_recv()
        r_top_right.wait_recv()
        r_top_diag.wait_recv()
        r_bot_diag.wait_recv()

        for d in (s_top_r, s_bot_r, s_bot_l, s_top_l, s_fwd_r, s_fwd_l):
            d.wait_send()

    return pl.pallas_call(
        body,
        out_shape=jax.ShapeDtypeStruct((N_DEV * m_per, n), jnp.bfloat16),
        in_specs=[pl.BlockSpec(memory_space=pltpu.VMEM)],
        out_specs=pl.BlockSpec(memory_space=pltpu.VMEM),
        scratch_shapes=[
            pltpu.SemaphoreType.DMA((6,)),
            pltpu.SemaphoreType.DMA((6,)),
        ],
        compiler_params=pltpu.CompilerParams(collective_id=0),
    )(x)
